# baseline (speedup 1.0000x reference)
import functools
import jax
import jax.numpy as jnp
from jax import lax
from jax.experimental import pallas as pl
from jax.experimental.pallas import tpu as pltpu
from jax.experimental.pallas import tpu_sc as plsc

D = H = W = 128
HW = H * W
NVOX = D * HW
NB = 2
NW = 32
CH = 128
VPW = NVOX // NW
NCHUNK = VPW // CH
G = CH // 16

_mesh = plsc.VectorSubcoreMesh(core_axis_name="c", subcore_axis_name="s")


@functools.partial(
    pl.kernel,
    mesh=_mesh,
    out_type=[jax.ShapeDtypeStruct((NVOX,), jnp.float32)] * (NB * 2),
    scratch_types=(
        [pltpu.VMEM((3 * CH,), jnp.float32)] * 2
        + [pltpu.VMEM((8, CH), jnp.int32)] * 2
        + [pltpu.VMEM((8, CH), jnp.float32)] * 2
        + [pltpu.VMEM((8 * CH,), jnp.float32)] * 2
        + [pltpu.VMEM((8 * CH,), jnp.float32)] * 2
        + [pltpu.VMEM((2 * CH,), jnp.float32)] * 2
        + [pltpu.SemaphoreType.DMA] * 8
    ),
)
def _sc_warp(v00, v01, v10, v11, d0z, d0y, d0x, d1z, d1y, d1x,
             o00, o01, o10, o11,
             dfa0, dfa1, idx0, idx1, wv0, wv1, ga0, ga1, gb0, gb1,
             ov0, ov1,
             dfsem0, dfsem1, gasem0, gasem1, gbsem0, gbsem1, osem0, osem1):
    wid = lax.axis_index("s") * 2 + lax.axis_index("c")
    lanes = lax.iota(jnp.int32, 16)
    dfa = (dfa0, dfa1)
    idx = (idx0, idx1)
    wv = (wv0, wv1)
    ga = (ga0, ga1)
    gb = (gb0, gb1)
    ov = (ov0, ov1)
    dfsem = (dfsem0, dfsem1)
    gasem = (gasem0, gasem1)
    gbsem = (gbsem0, gbsem1)
    osem = (osem0, osem1)

    def run_batch(vb0, vb1, dbz, dby, dbx, ob0, ob1, prime):
        def prefetch(s, c):
            p0 = wid * VPW + c * CH
            pltpu.async_copy(dbz.at[pl.ds(p0, CH)],
                             dfa[s].at[pl.ds(0, CH)], dfsem[s])
            pltpu.async_copy(dby.at[pl.ds(p0, CH)],
                             dfa[s].at[pl.ds(CH, CH)], dfsem[s])
            pltpu.async_copy(dbx.at[pl.ds(p0, CH)],
                             dfa[s].at[pl.ds(2 * CH, CH)], dfsem[s])

        def df_drain(s):
            pltpu.make_async_copy(dbz.at[pl.ds(0, 3 * CH)], dfa[s],
                                  dfsem[s]).wait()

        def compute(s, c):
            p0 = wid * VPW + c * CH
            for g in range(G):
                sl = pl.ds(g * 16, 16)
                p = p0 + g * 16 + lanes
                z = lax.shift_right_logical(p, 14)
                y = jnp.bitwise_and(lax.shift_right_logical(p, 7), 127)
                x = jnp.bitwise_and(p, 127)

                locz = jnp.minimum(jnp.maximum(
                    z.astype(jnp.float32) + dfa[s][pl.ds(g * 16, 16)],
                    0.0), 127.0)
                locy = jnp.minimum(jnp.maximum(
                    y.astype(jnp.float32) + dfa[s][pl.ds(CH + g * 16, 16)],
                    0.0), 127.0)
                locx = jnp.minimum(jnp.maximum(
                    x.astype(jnp.float32) + dfa[s][pl.ds(2 * CH + g * 16, 16)],
                    0.0), 127.0)

                az = jnp.minimum(locz.astype(jnp.int32), 126)
                ay = jnp.minimum(locy.astype(jnp.int32), 126)
                ax = jnp.minimum(locx.astype(jnp.int32), 126)
                tz = locz - az.astype(jnp.float32)
                ty = locy - ay.astype(jnp.float32)
                tx = locx - ax.astype(jnp.float32)
                uz = 1.0 - tz
                uy = 1.0 - ty
                ux = 1.0 - tx

                base = (lax.shift_left(az, 14) + lax.shift_left(ay, 7) + ax)
                w00 = uz * uy
                w01 = uz * ty
                w10 = tz * uy
                w11 = tz * ty

                idx[s][0, sl] = p
                idx[s][1, sl] = jnp.bitwise_and(p + 1 * 2048, NVOX - 1)
                idx[s][2, sl] = jnp.bitwise_and(p + 2 * 2048, NVOX - 1)
                idx[s][3, sl] = jnp.bitwise_and(p + 3 * 2048, NVOX - 1)
                idx[s][4, sl] = jnp.bitwise_and(p + 4 * 2048, NVOX - 1)
                idx[s][5, sl] = jnp.bitwise_and(p + 5 * 2048, NVOX - 1)
                idx[s][6, sl] = jnp.bitwise_and(p + 6 * 2048, NVOX - 1)
                idx[s][7, sl] = jnp.bitwise_and(p + 7 * 2048, NVOX - 1)
                wv[s][0, sl] = w00 * ux
                wv[s][1, sl] = w00 * tx
                wv[s][2, sl] = w01 * ux
                wv[s][3, sl] = w01 * tx
                wv[s][4, sl] = w10 * ux
                wv[s][5, sl] = w10 * tx
                wv[s][6, sl] = w11 * ux
                wv[s][7, sl] = w11 * tx

        def gather_enq(s):
            for k in range(8):
                pltpu.async_copy(vb0.at[idx[s].at[k]],
                                 ga[s].at[pl.ds(k * CH, CH)], gasem[s])
                pltpu.async_copy(vb1.at[idx[s].at[k]],
                                 gb[s].at[pl.ds(k * CH, CH)], gbsem[s])

        def stage_a(s, c, pf_c=None):
            df_drain(s)
            compute(s, c)
            gather_enq(s)
            if pf_c is not None:
                prefetch(s, pf_c)

        def out_drain(s):
            pltpu.make_async_copy(ob0.at[pl.ds(0, 2 * CH)], ov[s],
                                  osem[s]).wait()

        def stage_b(s, c):
            pltpu.make_async_copy(vb0.at[pl.ds(0, 8 * CH)], ga[s],
                                  gasem[s]).wait()
            pltpu.make_async_copy(vb1.at[pl.ds(0, 8 * CH)], gb[s],
                                  gbsem[s]).wait()
            out_drain(s)
            for g in range(G):
                sl = pl.ds(g * 16, 16)
                w0 = wv[s][0, sl]
                acc0 = w0 * ga[s][pl.ds(g * 16, 16)]
                acc1 = w0 * gb[s][pl.ds(g * 16, 16)]
                for k in range(1, 8):
                    wk = wv[s][k, sl]
                    acc0 = acc0 + wk * ga[s][pl.ds(k * CH + g * 16, 16)]
                    acc1 = acc1 + wk * gb[s][pl.ds(k * CH + g * 16, 16)]
                ov[s][pl.ds(g * 16, 16)] = acc0
                ov[s][pl.ds(CH + g * 16, 16)] = acc1
            p0 = wid * VPW + c * CH
            pltpu.async_copy(ov[s].at[pl.ds(0, CH)],
                             ob0.at[pl.ds(p0, CH)], osem[s])
            pltpu.async_copy(ov[s].at[pl.ds(CH, CH)],
                             ob1.at[pl.ds(p0, CH)], osem[s])

        if prime:
            for s in range(2):
                pltpu.async_copy(ov[s].at[pl.ds(0, CH)],
                                 ob0.at[pl.ds(wid * VPW, CH)], osem[s])
                pltpu.async_copy(ov[s].at[pl.ds(CH, CH)],
                                 ob1.at[pl.ds(wid * VPW, CH)], osem[s])

        prefetch(0, 0)
        prefetch(1, 1)
        stage_a(0, 0, 2)

        def body(i, carry):
            c1 = 2 * i + 1
            stage_a(1, c1, c1 + 2)
            stage_b(0, c1 - 1)
            c2 = 2 * i + 2
            stage_a(0, c2, jnp.minimum(c2 + 2, NCHUNK - 1))
            stage_b(1, c2 - 1)
            return carry

        lax.fori_loop(0, (NCHUNK - 2) // 2, body, 0)
        stage_a(1, NCHUNK - 1)
        stage_b(0, NCHUNK - 2)
        stage_b(1, NCHUNK - 1)
        df_drain(0)

    run_batch(v00, v01, d0z, d0y, d0x, o00, o01, True)
    run_batch(v10, v11, d1z, d1y, d1x, o10, o11, False)
    pltpu.make_async_copy(o10.at[pl.ds(0, 2 * CH)], ov0, osem0).wait()
    pltpu.make_async_copy(o10.at[pl.ds(0, 2 * CH)], ov1, osem1).wait()


def kernel(vol, df):
    v = vol.reshape(NB, 2, NVOX)
    d = df.reshape(NB, 3, NVOX)
    o00, o01, o10, o11 = _sc_warp(
        v[0, 0], v[0, 1], v[1, 0], v[1, 1],
        d[0, 0], d[0, 1], d[0, 2], d[1, 0], d[1, 1], d[1, 2])
    return jnp.stack([jnp.stack([o00, o01]), jnp.stack([o10, o11])]
                     ).reshape(NB, 2, D, H, W)


# speedup vs baseline: 5.0755x; 5.0755x over previous
import functools
import jax
import jax.numpy as jnp
from jax import lax
from jax.experimental import pallas as pl
from jax.experimental.pallas import tpu as pltpu
from jax.experimental.pallas import tpu_sc as plsc

D = H = W = 128
HW = H * W
NVOX = D * HW
NB = 2
NW = 32
BZ = BY = 8
SZ = 18
SY = 24
SYW = SY * W
MARGIN = 5
NBLK = (D // BZ) * (H // BY)
BPW = NBLK // NW
NROW = BZ * BY
CH = W
G = CH // 16

_mesh = plsc.VectorSubcoreMesh(core_axis_name="c", subcore_axis_name="s")


@functools.partial(
    pl.kernel,
    mesh=_mesh,
    compiler_params=pltpu.CompilerParams(needs_layout_passes=False),
    out_type=[jax.ShapeDtypeStruct((NB * NVOX,), jnp.float32)] * 2,
    scratch_types=(
        [pltpu.VMEM((SZ * SYW,), jnp.float32)] * 2
        + [pltpu.VMEM((3 * CH,), jnp.float32)] * 2
        + [pltpu.VMEM((2, CH), jnp.int32)]
        + [pltpu.VMEM((8, CH), jnp.float32)]
        + [pltpu.VMEM((1, CH), jnp.int32)]
        + [pltpu.VMEM((CH,), jnp.float32)] * 2
        + [pltpu.VMEM((2 * CH,), jnp.float32)] * 2
        + [pltpu.SemaphoreType.DMA] * 6
    ),
)
def _sc_warp(vc0, vc1, dz_, dy_, dx_, oc0, oc1,
             slab0, slab1, dfa0, dfa1, loci, wv, idxf, gf0, gf1,
             ov0, ov1,
             dfsem0, dfsem1, ssem, fsem, osem0, osem1):
    wid = lax.axis_index("s") * 2 + lax.axis_index("c")
    lanes = lax.iota(jnp.int32, 16)
    dfa = (dfa0, dfa1)
    ov = (ov0, ov1)
    dfsem = (dfsem0, dfsem1)
    osem = (osem0, osem1)
    vbf = (vc0, vc1)
    ob = (oc0, oc1)

    blk0 = wid * BPW
    off0 = (lax.shift_right_logical(blk0, 4) * BZ * HW
            + jnp.bitwise_and(blk0, 15) * BY * W)
    for s in range(2):
        pltpu.async_copy(ov[s].at[pl.ds(0, CH)],
                         oc0.at[pl.ds(off0, CH)], osem[s])
        pltpu.async_copy(ov[s].at[pl.ds(CH, CH)],
                         oc1.at[pl.ds(off0, CH)], osem[s])

    def run_block(boff, zb, yb):
        z0 = jnp.clip(zb - MARGIN, 0, D - SZ)
        y0 = pl.multiple_of(jnp.clip(yb - 8, 0, H - SY), 8)

        hs = []
        for c in range(2):
            for zi in range(SZ):
                hs.append(pltpu.async_copy(
                    vbf[c].at[pl.ds(boff + (z0 + zi) * HW + y0 * W, SYW)],
                    (slab0 if c == 0 else slab1).at[pl.ds(zi * SYW, SYW)],
                    ssem))

        def rowoff(t):
            z = zb + lax.shift_right_logical(t, 3)
            y = yb + jnp.bitwise_and(t, 7)
            return z, y, z * HW + y * W

        def prefetch(s, t):
            _, _, off = rowoff(t)
            pltpu.async_copy(dz_.at[pl.ds(boff + off, CH)],
                             dfa[s].at[pl.ds(0, CH)], dfsem[s])
            pltpu.async_copy(dy_.at[pl.ds(boff + off, CH)],
                             dfa[s].at[pl.ds(CH, CH)], dfsem[s])
            pltpu.async_copy(dx_.at[pl.ds(boff + off, CH)],
                             dfa[s].at[pl.ds(2 * CH, CH)], dfsem[s])

        def df_drain(s):
            pltpu.make_async_copy(dz_.at[pl.ds(0, 3 * CH)], dfa[s],
                                  dfsem[s]).wait()

        def out_drain(s):
            pltpu.make_async_copy(oc0.at[pl.ds(0, 2 * CH)], ov[s],
                                  osem[s]).wait()

        def do_row(s, t):
            z, y, off = rowoff(t)
            df_drain(s)
            zf = z.astype(jnp.float32)
            yf = y.astype(jnp.float32)
            bad = 0
            for g in range(G):
                sl = pl.ds(g * 16, 16)
                x = g * 16 + lanes
                locz = jnp.minimum(jnp.maximum(
                    zf + dfa[s][pl.ds(g * 16, 16)], 0.0), 127.0)
                locy = jnp.minimum(jnp.maximum(
                    yf + dfa[s][pl.ds(CH + g * 16, 16)], 0.0), 127.0)
                locx = jnp.minimum(jnp.maximum(
                    x.astype(jnp.float32) + dfa[s][pl.ds(2 * CH + g * 16, 16)],
                    0.0), 127.0)
                az = jnp.minimum(locz.astype(jnp.int32), 126)
                ay = jnp.minimum(locy.astype(jnp.int32), 126)
                ax = jnp.minimum(locx.astype(jnp.int32), 126)
                tz = locz - az.astype(jnp.float32)
                ty = locy - ay.astype(jnp.float32)
                tx = locx - ax.astype(jnp.float32)
                uz = 1.0 - tz
                uy = 1.0 - ty
                ux = 1.0 - tx
                zi = az - z0
                yi = ay - y0
                ok = ((zi >= 0) & (zi <= SZ - 2)
                      & (yi >= 0) & (yi <= SY - 2))
                bad = bad + (16 - jnp.sum(ok.astype(jnp.int32)))
                loci[0, sl] = zi * SYW + yi * W + ax
                loci[1, sl] = (boff + lax.shift_left(az, 14)
                               + lax.shift_left(ay, 7) + ax)
                w00 = uz * uy
                w01 = uz * ty
                w10 = tz * uy
                w11 = tz * ty
                wv[0, sl] = w00 * ux
                wv[1, sl] = w00 * tx
                wv[2, sl] = w01 * ux
                wv[3, sl] = w01 * tx
                wv[4, sl] = w10 * ux
                wv[5, sl] = w10 * tx
                wv[6, sl] = w11 * ux
                wv[7, sl] = w11 * tx

            out_drain(s)

            @pl.when(bad == 0)
            def _hot():
                for g in range(G):
                    sl = pl.ds(g * 16, 16)
                    fb = loci[0, sl]
                    acc0 = wv[0, sl] * plsc.load_gather(slab0, [fb])
                    acc1 = wv[0, sl] * plsc.load_gather(slab1, [fb])
                    for k, o_ in enumerate(
                            (1, W, W + 1, SYW, SYW + 1, SYW + W, SYW + W + 1),
                            start=1):
                        wk = wv[k, sl]
                        fo = fb + o_
                        acc0 = acc0 + wk * plsc.load_gather(slab0, [fo])
                        acc1 = acc1 + wk * plsc.load_gather(slab1, [fo])
                    ov[s][pl.ds(g * 16, 16)] = acc0
                    ov[s][pl.ds(CH + g * 16, 16)] = acc1

            @pl.when(bad != 0)
            def _cold():
                for k, o_ in enumerate((0, 1, 128, 129, 16384, 16385,
                                        16512, 16513)):
                    for g in range(G):
                        sl = pl.ds(g * 16, 16)
                        idxf[0, sl] = loci[1, sl] + o_
                    h0 = pltpu.async_copy(vc0.at[idxf.at[0]], gf0, fsem)
                    h1 = pltpu.async_copy(vc1.at[idxf.at[0]], gf1, fsem)
                    h0.wait()
                    h1.wait()
                    for g in range(G):
                        sl = pl.ds(g * 16, 16)
                        sl1 = pl.ds(CH + g * 16, 16)
                        wk = wv[k, sl]
                        if k == 0:
                            ov[s][sl] = wk * gf0[sl]
                            ov[s][sl1] = wk * gf1[sl]
                        else:
                            ov[s][sl] = ov[s][sl] + wk * gf0[sl]
                            ov[s][sl1] = ov[s][sl1] + wk * gf1[sl]

            pltpu.async_copy(ov[s].at[pl.ds(0, CH)],
                             oc0.at[pl.ds(boff + off, CH)], osem[s])
            pltpu.async_copy(ov[s].at[pl.ds(CH, CH)],
                             oc1.at[pl.ds(boff + off, CH)], osem[s])

        prefetch(0, 0)
        prefetch(1, 1)
        for h in hs:
            h.wait()

        def rows(i, carry):
            t0 = 2 * i
            do_row(0, t0)
            prefetch(0, jnp.minimum(t0 + 2, NROW - 1))
            do_row(1, t0 + 1)
            prefetch(1, jnp.minimum(t0 + 3, NROW - 1))
            return carry

        lax.fori_loop(0, NROW // 2, rows, 0)
        df_drain(0)
        df_drain(1)

    def blocks(i, carry):
        boff = lax.shift_right_logical(i, 3) * NVOX
        blk = wid * BPW + jnp.bitwise_and(i, 7)
        zb = lax.shift_right_logical(blk, 4) * BZ
        yb = jnp.bitwise_and(blk, 15) * BY
        run_block(boff, zb, yb)
        return carry

    lax.fori_loop(0, NB * BPW, blocks, 0)

    pltpu.make_async_copy(oc0.at[pl.ds(0, 2 * CH)], ov0, osem0).wait()
    pltpu.make_async_copy(oc0.at[pl.ds(0, 2 * CH)], ov1, osem1).wait()


def kernel(vol, df):
    vcm = vol.transpose(1, 0, 2, 3, 4).reshape(2, NB * NVOX)
    dcm = df.transpose(1, 0, 2, 3, 4).reshape(3, NB * NVOX)
    oc0, oc1 = _sc_warp(vcm[0], vcm[1], dcm[0], dcm[1], dcm[2])
    return (jnp.stack([oc0, oc1]).reshape(2, NB, D, H, W)
            .transpose(1, 0, 2, 3, 4))


# speedup vs baseline: 7.1948x; 1.4176x over previous
import functools
import jax
import jax.numpy as jnp
from jax import lax
from jax.experimental import pallas as pl
from jax.experimental.pallas import tpu as pltpu
from jax.experimental.pallas import tpu_sc as plsc

D = H = W = 128
HW = H * W
NVOX = D * HW
NB = 2
NW = 32
BZ = BY = 8
SZ = 18
SY = 24
SYW = SY * W
MARGIN = 5
NBLK = (D // BZ) * (H // BY)
BPW = NBLK // NW
NROW = BZ * BY
CH = W
G = CH // 16

_mesh = plsc.VectorSubcoreMesh(core_axis_name="c", subcore_axis_name="s")


@functools.partial(
    pl.kernel,
    mesh=_mesh,
    compiler_params=pltpu.CompilerParams(needs_layout_passes=False),
    out_type=jax.ShapeDtypeStruct((NB * 2 * NVOX,), jnp.float32),
    scratch_types=(
        [pltpu.VMEM((SZ * SYW,), jnp.float32)] * 2
        + [pltpu.VMEM((3 * CH,), jnp.float32)] * 2
        + [pltpu.VMEM((2, CH), jnp.int32)]
        + [pltpu.VMEM((8, CH), jnp.float32)]
        + [pltpu.VMEM((1, CH), jnp.int32)]
        + [pltpu.VMEM((CH,), jnp.float32)] * 2
        + [pltpu.VMEM((2 * CH,), jnp.float32)] * 2
        + [pltpu.SemaphoreType.DMA] * 6
    ),
)
def _sc_warp(vf, dff, out,
             slab0, slab1, dfa0, dfa1, loci, wv, idxf, gf0, gf1,
             ov0, ov1,
             dfsem0, dfsem1, ssem, fsem, osem0, osem1):
    wid = lax.axis_index("s") * 2 + lax.axis_index("c")
    lanes = lax.iota(jnp.int32, 16)
    dfa = (dfa0, dfa1)
    ov = (ov0, ov1)
    dfsem = (dfsem0, dfsem1)
    osem = (osem0, osem1)

    blk0 = wid * BPW
    off0 = (lax.shift_right_logical(blk0, 4) * BZ * HW
            + jnp.bitwise_and(blk0, 15) * BY * W)
    for s in range(2):
        pltpu.async_copy(ov[s].at[pl.ds(0, CH)],
                         out.at[pl.ds(off0, CH)], osem[s])
        pltpu.async_copy(ov[s].at[pl.ds(CH, CH)],
                         out.at[pl.ds(NVOX + off0, CH)], osem[s])

    def run_block(bv, bd, zb, yb):
        z0 = jnp.clip(zb - MARGIN, 0, D - SZ)
        y0 = pl.multiple_of(jnp.clip(yb - 8, 0, H - SY), 8)

        hs = []
        for c in range(2):
            for zi in range(SZ):
                hs.append(pltpu.async_copy(
                    vf.at[pl.ds(bv + c * NVOX + (z0 + zi) * HW + y0 * W,
                                SYW)],
                    (slab0 if c == 0 else slab1).at[pl.ds(zi * SYW, SYW)],
                    ssem))

        def rowoff(t):
            z = zb + lax.shift_right_logical(t, 3)
            y = yb + jnp.bitwise_and(t, 7)
            return z, y, z * HW + y * W

        def prefetch(s, t):
            _, _, off = rowoff(t)
            pltpu.async_copy(dff.at[pl.ds(bd + off, CH)],
                             dfa[s].at[pl.ds(0, CH)], dfsem[s])
            pltpu.async_copy(dff.at[pl.ds(bd + NVOX + off, CH)],
                             dfa[s].at[pl.ds(CH, CH)], dfsem[s])
            pltpu.async_copy(dff.at[pl.ds(bd + 2 * NVOX + off, CH)],
                             dfa[s].at[pl.ds(2 * CH, CH)], dfsem[s])

        def df_drain(s):
            pltpu.make_async_copy(dff.at[pl.ds(0, 3 * CH)], dfa[s],
                                  dfsem[s]).wait()

        def out_drain(s):
            pltpu.make_async_copy(out.at[pl.ds(0, 2 * CH)], ov[s],
                                  osem[s]).wait()

        def do_row(s, t):
            z, y, off = rowoff(t)
            df_drain(s)
            zf = z.astype(jnp.float32)
            yf = y.astype(jnp.float32)
            bad = 0
            for g in range(G):
                sl = pl.ds(g * 16, 16)
                x = g * 16 + lanes
                locz = jnp.minimum(jnp.maximum(
                    zf + dfa[s][pl.ds(g * 16, 16)], 0.0), 127.0)
                locy = jnp.minimum(jnp.maximum(
                    yf + dfa[s][pl.ds(CH + g * 16, 16)], 0.0), 127.0)
                locx = jnp.minimum(jnp.maximum(
                    x.astype(jnp.float32) + dfa[s][pl.ds(2 * CH + g * 16, 16)],
                    0.0), 127.0)
                az = jnp.minimum(locz.astype(jnp.int32), 126)
                ay = jnp.minimum(locy.astype(jnp.int32), 126)
                ax = jnp.minimum(locx.astype(jnp.int32), 126)
                tz = locz - az.astype(jnp.float32)
                ty = locy - ay.astype(jnp.float32)
                tx = locx - ax.astype(jnp.float32)
                uz = 1.0 - tz
                uy = 1.0 - ty
                ux = 1.0 - tx
                zi = az - z0
                yi = ay - y0
                ok = ((zi >= 0) & (zi <= SZ - 2)
                      & (yi >= 0) & (yi <= SY - 2))
                bad = bad + (16 - jnp.sum(ok.astype(jnp.int32)))
                loci[0, sl] = zi * SYW + yi * W + ax
                loci[1, sl] = (bv + lax.shift_left(az, 14)
                               + lax.shift_left(ay, 7) + ax)
                w00 = uz * uy
                w01 = uz * ty
                w10 = tz * uy
                w11 = tz * ty
                wv[0, sl] = w00 * ux
                wv[1, sl] = w00 * tx
                wv[2, sl] = w01 * ux
                wv[3, sl] = w01 * tx
                wv[4, sl] = w10 * ux
                wv[5, sl] = w10 * tx
                wv[6, sl] = w11 * ux
                wv[7, sl] = w11 * tx

            out_drain(s)

            @pl.when(bad == 0)
            def _hot():
                for g in range(G):
                    sl = pl.ds(g * 16, 16)
                    fb = loci[0, sl]
                    acc0 = wv[0, sl] * plsc.load_gather(slab0, [fb])
                    acc1 = wv[0, sl] * plsc.load_gather(slab1, [fb])
                    for k, o_ in enumerate(
                            (1, W, W + 1, SYW, SYW + 1, SYW + W, SYW + W + 1),
                            start=1):
                        wk = wv[k, sl]
                        fo = fb + o_
                        acc0 = acc0 + wk * plsc.load_gather(slab0, [fo])
                        acc1 = acc1 + wk * plsc.load_gather(slab1, [fo])
                    ov[s][pl.ds(g * 16, 16)] = acc0
                    ov[s][pl.ds(CH + g * 16, 16)] = acc1

            @pl.when(bad != 0)
            def _cold():
                for k, o_ in enumerate((0, 1, 128, 129, 16384, 16385,
                                        16512, 16513)):
                    for g in range(G):
                        sl = pl.ds(g * 16, 16)
                        idxf[0, sl] = loci[1, sl] + o_
                    h0 = pltpu.async_copy(vf.at[idxf.at[0]], gf0, fsem)
                    h0.wait()
                    for g in range(G):
                        sl = pl.ds(g * 16, 16)
                        idxf[0, sl] = loci[1, sl] + (o_ + NVOX)
                    h1 = pltpu.async_copy(vf.at[idxf.at[0]], gf1, fsem)
                    h1.wait()
                    for g in range(G):
                        sl = pl.ds(g * 16, 16)
                        sl1 = pl.ds(CH + g * 16, 16)
                        wk = wv[k, sl]
                        if k == 0:
                            ov[s][sl] = wk * gf0[sl]
                            ov[s][sl1] = wk * gf1[sl]
                        else:
                            ov[s][sl] = ov[s][sl] + wk * gf0[sl]
                            ov[s][sl1] = ov[s][sl1] + wk * gf1[sl]

            pltpu.async_copy(ov[s].at[pl.ds(0, CH)],
                             out.at[pl.ds(bv + off, CH)], osem[s])
            pltpu.async_copy(ov[s].at[pl.ds(CH, CH)],
                             out.at[pl.ds(bv + NVOX + off, CH)], osem[s])

        prefetch(0, 0)
        prefetch(1, 1)
        for h in hs:
            h.wait()

        def rows(i, carry):
            t0 = 2 * i
            do_row(0, t0)
            prefetch(0, jnp.minimum(t0 + 2, NROW - 1))
            do_row(1, t0 + 1)
            prefetch(1, jnp.minimum(t0 + 3, NROW - 1))
            return carry

        lax.fori_loop(0, NROW // 2, rows, 0)
        df_drain(0)
        df_drain(1)

    def blocks(i, carry):
        b = lax.shift_right_logical(i, 3)
        blk = wid * BPW + jnp.bitwise_and(i, 7)
        zb = lax.shift_right_logical(blk, 4) * BZ
        yb = jnp.bitwise_and(blk, 15) * BY
        run_block(b * (2 * NVOX), b * (3 * NVOX), zb, yb)
        return carry

    lax.fori_loop(0, NB * BPW, blocks, 0)

    pltpu.make_async_copy(out.at[pl.ds(0, 2 * CH)], ov0, osem0).wait()
    pltpu.make_async_copy(out.at[pl.ds(0, 2 * CH)], ov1, osem1).wait()


def kernel(vol, df):
    out = _sc_warp(vol.reshape(NB * 2 * NVOX), df.reshape(NB * 3 * NVOX))
    return out.reshape(NB, 2, D, H, W)


# speedup vs baseline: 7.5087x; 1.0436x over previous
import functools
import jax
import jax.numpy as jnp
from jax import lax
from jax.experimental import pallas as pl
from jax.experimental.pallas import tpu as pltpu
from jax.experimental.pallas import tpu_sc as plsc

D = H = W = 128
HW = H * W
NVOX = D * HW
NB = 2
NW = 32
BZ = BY = 8
SZ = 18
SY = 24
SYW = SY * W
MARGIN = 5
NBLK = (D // BZ) * (H // BY)
BPW = NBLK // NW
NROW = BZ * BY
CH = W
G = CH // 16

_mesh = plsc.VectorSubcoreMesh(core_axis_name="c", subcore_axis_name="s")


@functools.partial(
    pl.kernel,
    mesh=_mesh,
    compiler_params=pltpu.CompilerParams(needs_layout_passes=False),
    out_type=jax.ShapeDtypeStruct((NB * 2 * NVOX,), jnp.float32),
    scratch_types=(
        [pltpu.VMEM((SZ * SYW,), jnp.float32)] * 2
        + [pltpu.VMEM((3 * CH,), jnp.float32)] * 2
        + [pltpu.VMEM((2, CH), jnp.int32)]
        + [pltpu.VMEM((CH,), jnp.float32)]
        + [pltpu.VMEM((8, CH), jnp.float32)]
        + [pltpu.VMEM((1, CH), jnp.int32)]
        + [pltpu.VMEM((CH,), jnp.float32)] * 2
        + [pltpu.VMEM((2 * CH,), jnp.float32)] * 2
        + [pltpu.SemaphoreType.DMA] * 6
    ),
)
def _sc_warp(vf, dff, out,
             slab0, slab1, dfa0, dfa1, loci, xfv, wv, idxf, gf0, gf1,
             ov0, ov1,
             dfsem0, dfsem1, ssem, fsem, osem0, osem1):
    wid = lax.axis_index("s") * 2 + lax.axis_index("c")
    lanes = lax.iota(jnp.int32, 16)
    for g in range(G):
        xfv[pl.ds(g * 16, 16)] = (g * 16 + lanes).astype(jnp.float32)
    dfa = (dfa0, dfa1)
    ov = (ov0, ov1)
    dfsem = (dfsem0, dfsem1)
    osem = (osem0, osem1)

    blk0 = wid * BPW
    off0 = (lax.shift_right_logical(blk0, 4) * BZ * HW
            + jnp.bitwise_and(blk0, 15) * BY * W)
    for s in range(2):
        pltpu.async_copy(ov[s].at[pl.ds(0, CH)],
                         out.at[pl.ds(off0, CH)], osem[s])
        pltpu.async_copy(ov[s].at[pl.ds(CH, CH)],
                         out.at[pl.ds(NVOX + off0, CH)], osem[s])

    def run_block(bv, bd, zb, yb):
        z0 = jnp.clip(zb - MARGIN, 0, D - SZ)
        y0 = pl.multiple_of(jnp.clip(yb - 8, 0, H - SY), 8)

        hs = []
        for c in range(2):
            for zi in range(SZ):
                hs.append(pltpu.async_copy(
                    vf.at[pl.ds(bv + c * NVOX + (z0 + zi) * HW + y0 * W,
                                SYW)],
                    (slab0 if c == 0 else slab1).at[pl.ds(zi * SYW, SYW)],
                    ssem))

        def rowoff(t):
            z = zb + lax.shift_right_logical(t, 3)
            y = yb + jnp.bitwise_and(t, 7)
            return z, y, z * HW + y * W

        def prefetch(s, t):
            _, _, off = rowoff(t)
            pltpu.async_copy(dff.at[pl.ds(bd + off, CH)],
                             dfa[s].at[pl.ds(0, CH)], dfsem[s])
            pltpu.async_copy(dff.at[pl.ds(bd + NVOX + off, CH)],
                             dfa[s].at[pl.ds(CH, CH)], dfsem[s])
            pltpu.async_copy(dff.at[pl.ds(bd + 2 * NVOX + off, CH)],
                             dfa[s].at[pl.ds(2 * CH, CH)], dfsem[s])

        def df_drain(s):
            pltpu.make_async_copy(dff.at[pl.ds(0, 3 * CH)], dfa[s],
                                  dfsem[s]).wait()

        def out_drain(s):
            pltpu.make_async_copy(out.at[pl.ds(0, 2 * CH)], ov[s],
                                  osem[s]).wait()

        def do_row(s, t):
            z, y, off = rowoff(t)
            df_drain(s)
            out_drain(s)
            zf = z.astype(jnp.float32)
            yf = y.astype(jnp.float32)
            bad = 0
            for g in range(G):
                sl = pl.ds(g * 16, 16)
                locz = jnp.minimum(jnp.maximum(
                    zf + dfa[s][pl.ds(g * 16, 16)], 0.0), 127.0)
                locy = jnp.minimum(jnp.maximum(
                    yf + dfa[s][pl.ds(CH + g * 16, 16)], 0.0), 127.0)
                locx = jnp.minimum(jnp.maximum(
                    xfv[sl] + dfa[s][pl.ds(2 * CH + g * 16, 16)],
                    0.0), 127.0)
                az = jnp.minimum(locz.astype(jnp.int32), 126)
                ay = jnp.minimum(locy.astype(jnp.int32), 126)
                ax = jnp.minimum(locx.astype(jnp.int32), 126)
                tz = locz - az.astype(jnp.float32)
                ty = locy - ay.astype(jnp.float32)
                tx = locx - ax.astype(jnp.float32)
                uz = 1.0 - tz
                uy = 1.0 - ty
                ux = 1.0 - tx
                zi = az - z0
                yi = ay - y0
                ok = ((zi >= 0) & (zi <= SZ - 2)
                      & (yi >= 0) & (yi <= SY - 2))
                bad = bad + (16 - jnp.sum(ok.astype(jnp.int32)))
                fb = jnp.clip(zi * SYW + yi * W + ax, 0,
                              SZ * SYW - SYW - W - 2)
                w00 = uz * uy
                w01 = uz * ty
                w10 = tz * uy
                w11 = tz * ty
                wk = w00 * ux
                acc0 = wk * plsc.load_gather(slab0, [fb])
                acc1 = wk * plsc.load_gather(slab1, [fb])
                for wgt, o_ in ((w00 * tx, 1),
                                (w01 * ux, W), (w01 * tx, W + 1),
                                (w10 * ux, SYW), (w10 * tx, SYW + 1),
                                (w11 * ux, SYW + W), (w11 * tx, SYW + W + 1)):
                    fo = fb + o_
                    acc0 = acc0 + wgt * plsc.load_gather(slab0, [fo])
                    acc1 = acc1 + wgt * plsc.load_gather(slab1, [fo])
                ov[s][pl.ds(g * 16, 16)] = acc0
                ov[s][pl.ds(CH + g * 16, 16)] = acc1

            @pl.when(bad != 0)
            def _cold():
                for g in range(G):
                    sl = pl.ds(g * 16, 16)
                    locz = jnp.minimum(jnp.maximum(
                        zf + dfa[s][pl.ds(g * 16, 16)], 0.0), 127.0)
                    locy = jnp.minimum(jnp.maximum(
                        yf + dfa[s][pl.ds(CH + g * 16, 16)], 0.0), 127.0)
                    locx = jnp.minimum(jnp.maximum(
                        xfv[sl] + dfa[s][pl.ds(2 * CH + g * 16, 16)],
                        0.0), 127.0)
                    az = jnp.minimum(locz.astype(jnp.int32), 126)
                    ay = jnp.minimum(locy.astype(jnp.int32), 126)
                    ax = jnp.minimum(locx.astype(jnp.int32), 126)
                    tz = locz - az.astype(jnp.float32)
                    ty = locy - ay.astype(jnp.float32)
                    tx = locx - ax.astype(jnp.float32)
                    uz = 1.0 - tz
                    uy = 1.0 - ty
                    ux = 1.0 - tx
                    loci[1, sl] = (bv + lax.shift_left(az, 14)
                                   + lax.shift_left(ay, 7) + ax)
                    w00 = uz * uy
                    w01 = uz * ty
                    w10 = tz * uy
                    w11 = tz * ty
                    wv[0, sl] = w00 * ux
                    wv[1, sl] = w00 * tx
                    wv[2, sl] = w01 * ux
                    wv[3, sl] = w01 * tx
                    wv[4, sl] = w10 * ux
                    wv[5, sl] = w10 * tx
                    wv[6, sl] = w11 * ux
                    wv[7, sl] = w11 * tx
                for k, o_ in enumerate((0, 1, 128, 129, 16384, 16385,
                                        16512, 16513)):
                    for g in range(G):
                        sl = pl.ds(g * 16, 16)
                        idxf[0, sl] = loci[1, sl] + o_
                    h0 = pltpu.async_copy(vf.at[idxf.at[0]], gf0, fsem)
                    h0.wait()
                    for g in range(G):
                        sl = pl.ds(g * 16, 16)
                        idxf[0, sl] = loci[1, sl] + (o_ + NVOX)
                    h1 = pltpu.async_copy(vf.at[idxf.at[0]], gf1, fsem)
                    h1.wait()
                    for g in range(G):
                        sl = pl.ds(g * 16, 16)
                        sl1 = pl.ds(CH + g * 16, 16)
                        wk = wv[k, sl]
                        if k == 0:
                            ov[s][sl] = wk * gf0[sl]
                            ov[s][sl1] = wk * gf1[sl]
                        else:
                            ov[s][sl] = ov[s][sl] + wk * gf0[sl]
                            ov[s][sl1] = ov[s][sl1] + wk * gf1[sl]

            pltpu.async_copy(ov[s].at[pl.ds(0, CH)],
                             out.at[pl.ds(bv + off, CH)], osem[s])
            pltpu.async_copy(ov[s].at[pl.ds(CH, CH)],
                             out.at[pl.ds(bv + NVOX + off, CH)], osem[s])

        prefetch(0, 0)
        prefetch(1, 1)
        for h in hs:
            h.wait()

        def rows(i, carry):
            t0 = 2 * i
            do_row(0, t0)
            prefetch(0, jnp.minimum(t0 + 2, NROW - 1))
            do_row(1, t0 + 1)
            prefetch(1, jnp.minimum(t0 + 3, NROW - 1))
            return carry

        lax.fori_loop(0, NROW // 2, rows, 0)
        df_drain(0)
        df_drain(1)

    def blocks(i, carry):
        b = lax.shift_right_logical(i, 3)
        blk = wid * BPW + jnp.bitwise_and(i, 7)
        zb = lax.shift_right_logical(blk, 4) * BZ
        yb = jnp.bitwise_and(blk, 15) * BY
        run_block(b * (2 * NVOX), b * (3 * NVOX), zb, yb)
        return carry

    lax.fori_loop(0, NB * BPW, blocks, 0)

    pltpu.make_async_copy(out.at[pl.ds(0, 2 * CH)], ov0, osem0).wait()
    pltpu.make_async_copy(out.at[pl.ds(0, 2 * CH)], ov1, osem1).wait()


def kernel(vol, df):
    out = _sc_warp(vol.reshape(NB * 2 * NVOX), df.reshape(NB * 3 * NVOX))
    return out.reshape(NB, 2, D, H, W)
